# baseline (speedup 1.0000x reference)
import functools

import jax
import jax.numpy as jnp
from jax import lax
from jax.experimental import pallas as pl
from jax.experimental.pallas import tpu as pltpu
from jax.experimental.pallas import tpu_sc as plsc

D_MODEL = 64
NC = 2
NS = 16
NW = NC * NS
CHUNK = 512
GATHER_BATCH = 128
NGB = CHUNK // GATHER_BATCH


def _positional_encoding(seq_len, d_model):
    pos = jnp.arange(0, seq_len, dtype=jnp.float32)[:, None]
    dim = jnp.arange(0, d_model, dtype=jnp.float32)
    result = jnp.zeros((seq_len, d_model), dtype=jnp.float32)
    sin_part = jnp.sin(pos / (10000.0 ** (dim[0::2] / d_model)))
    cos_part = jnp.cos(pos / (10000.0 ** (dim[1::2] / d_model)))
    result = result.at[:, 0::2].set(sin_part)
    result = result.at[:, 1::2].set(cos_part)
    return result


@functools.partial(jax.jit, static_argnames=("n_rows", "seq_len", "n_chunks"))
def _run(x_idx, pe_ext, table, n_rows, seq_len, n_chunks):
    mesh = plsc.VectorSubcoreMesh(
        core_axis_name="c", subcore_axis_name="s", num_cores=NC,
        num_subcores=NS)
    per_w = n_rows // NW
    pe_rows = pe_ext.shape[0]

    @functools.partial(
        pl.kernel,
        out_type=jax.ShapeDtypeStruct((n_rows, D_MODEL), jnp.float32),
        mesh=mesh,
        scratch_types=[
            pltpu.VMEM((pe_rows, D_MODEL), jnp.float32),
            pltpu.VMEM((NGB, GATHER_BATCH), jnp.int32),
            pltpu.VMEM((CHUNK, D_MODEL), jnp.float32),
            pltpu.SemaphoreType.DMA,
        ],
        compiler_params=pltpu.CompilerParams(use_tc_tiling_on_sc=False),
    )
    def k(x_hbm, pe_hbm, table_hbm, out_hbm, pe_v, idx_v, rows_v, gsem):
        cid = lax.axis_index("c")
        sid = lax.axis_index("s")
        wid = sid * NC + cid
        pltpu.sync_copy(pe_hbm, pe_v)

        def chunk_body(c, _):
            base = wid * per_w + c * CHUNK
            pos0 = lax.rem(c * CHUNK, seq_len)
            pltpu.sync_copy(x_hbm.at[wid * n_chunks + c], idx_v)
            cps = [
                pltpu.async_copy(
                    table_hbm.at[idx_v.at[j]],
                    rows_v.at[pl.ds(j * GATHER_BATCH, GATHER_BATCH)],
                    gsem,
                )
                for j in range(NGB)
            ]
            for cp in cps:
                cp.wait()

            def add_body(i, _):
                r = pos0 + i
                for j in range(D_MODEL // 16):
                    sl = pl.ds(j * 16, 16)
                    rows_v[i, sl] = rows_v[i, sl] + pe_v[r, sl]
                return ()

            lax.fori_loop(0, CHUNK, add_body, (), unroll=4)
            pltpu.sync_copy(rows_v, out_hbm.at[pl.ds(base, CHUNK)])
            return ()

        lax.fori_loop(0, n_chunks, chunk_body, ())

    return k(x_idx, pe_ext, table)


def kernel(x, table):
    b, s = x.shape
    n_rows = b * s
    per_w = n_rows // NW
    n_chunks = per_w // CHUNK
    xf = x.astype(jnp.int32).reshape(NW * n_chunks, NGB, GATHER_BATCH)
    pe = _positional_encoding(s, D_MODEL)
    reps = (s + CHUNK) // s + 1
    pe_ext = jnp.tile(pe, (reps, 1))[: s + CHUNK, :]
    out = _run(xf, pe_ext, table, n_rows, s, n_chunks)
    return out.reshape(b, s, D_MODEL)


# speedup vs baseline: 1.3411x; 1.3411x over previous
import functools

import jax
import jax.numpy as jnp
from jax import lax
from jax.experimental import pallas as pl
from jax.experimental.pallas import tpu as pltpu
from jax.experimental.pallas import tpu_sc as plsc

D_MODEL = 64
NC = 2
NS = 16
NW = NC * NS
CHUNK = 512
GATHER_BATCH = 128
NGB = CHUNK // GATHER_BATCH


def _positional_encoding(seq_len, d_model):
    pos = jnp.arange(0, seq_len, dtype=jnp.float32)[:, None]
    dim = jnp.arange(0, d_model, dtype=jnp.float32)
    result = jnp.zeros((seq_len, d_model), dtype=jnp.float32)
    sin_part = jnp.sin(pos / (10000.0 ** (dim[0::2] / d_model)))
    cos_part = jnp.cos(pos / (10000.0 ** (dim[1::2] / d_model)))
    result = result.at[:, 0::2].set(sin_part)
    result = result.at[:, 1::2].set(cos_part)
    return result


@functools.partial(jax.jit, static_argnames=("n_rows", "seq_len", "n_chunks"))
def _run(x_idx, pe_ext, table, n_rows, seq_len, n_chunks):
    mesh = plsc.VectorSubcoreMesh(
        core_axis_name="c", subcore_axis_name="s", num_cores=NC,
        num_subcores=NS)
    per_w = n_rows // NW
    pe_rows = pe_ext.shape[0]
    n_half = n_chunks // 2

    @functools.partial(
        pl.kernel,
        out_type=jax.ShapeDtypeStruct((n_rows, D_MODEL), jnp.float32),
        mesh=mesh,
        scratch_types=[
            pltpu.VMEM((pe_rows, D_MODEL), jnp.float32),
            pltpu.VMEM((NGB, GATHER_BATCH), jnp.int32),
            pltpu.VMEM((NGB, GATHER_BATCH), jnp.int32),
            pltpu.VMEM((CHUNK, D_MODEL), jnp.float32),
            pltpu.VMEM((CHUNK, D_MODEL), jnp.float32),
            pltpu.SemaphoreType.DMA,
            pltpu.SemaphoreType.DMA,
            pltpu.SemaphoreType.DMA,
            pltpu.SemaphoreType.DMA,
        ],
        compiler_params=pltpu.CompilerParams(use_tc_tiling_on_sc=False),
    )
    def k(x_hbm, pe_hbm, table_hbm, out_hbm, pe_v, idx0, idx1, rows0, rows1,
          gsem0, gsem1, ssem0, ssem1):
        cid = lax.axis_index("c")
        sid = lax.axis_index("s")
        wid = sid * NC + cid
        idx_b = (idx0, idx1)
        rows_b = (rows0, rows1)
        gsem = (gsem0, gsem1)
        ssem = (ssem0, ssem1)
        pltpu.sync_copy(pe_hbm, pe_v)

        def fire(c, b):
            pltpu.sync_copy(x_hbm.at[wid * n_chunks + c], idx_b[b])
            for j in range(NGB):
                pltpu.async_copy(
                    table_hbm.at[idx_b[b].at[j]],
                    rows_b[b].at[pl.ds(j * GATHER_BATCH, GATHER_BATCH)],
                    gsem[b],
                )

        def drain_gather(b):
            for j in range(NGB):
                pltpu.make_async_copy(
                    table_hbm.at[idx_b[b].at[j]],
                    rows_b[b].at[pl.ds(j * GATHER_BATCH, GATHER_BATCH)],
                    gsem[b],
                ).wait()

        def add_pe(b, c):
            pos0 = lax.rem(c * CHUNK, seq_len)
            rv = rows_b[b]

            @plsc.parallel_loop(0, CHUNK, unroll=4)
            def _(i):
                r = pos0 + i
                for j in range(D_MODEL // 16):
                    sl = pl.ds(j * 16, 16)
                    rv[i, sl] = rv[i, sl] + pe_v[r, sl]

        def store(c, b):
            pltpu.async_copy(
                rows_b[b], out_hbm.at[pl.ds(wid * per_w + c * CHUNK, CHUNK)],
                ssem[b])

        def drain_store(c, b):
            pltpu.make_async_copy(
                rows_b[b], out_hbm.at[pl.ds(wid * per_w + c * CHUNK, CHUNK)],
                ssem[b]).wait()

        fire(0, 0)

        def pair_body(kk, _):
            c0 = 2 * kk
            c1 = c0 + 1

            @pl.when(kk > 0)
            def _():
                drain_store(c1 - 2, 1)

            fire(c1, 1)
            drain_gather(0)
            add_pe(0, c0)
            store(c0, 0)

            @pl.when(kk < n_half - 1)
            def _():
                drain_store(c0, 0)
                fire(c0 + 2, 0)

            drain_gather(1)
            add_pe(1, c1)
            store(c1, 1)
            return ()

        lax.fori_loop(0, n_half, pair_body, ())
        drain_store(n_chunks - 2, 0)
        drain_store(n_chunks - 1, 1)

    return k(x_idx, pe_ext, table)


def kernel(x, table):
    b, s = x.shape
    n_rows = b * s
    per_w = n_rows // NW
    n_chunks = per_w // CHUNK
    xf = x.astype(jnp.int32).reshape(NW * n_chunks, NGB, GATHER_BATCH)
    pe = _positional_encoding(s, D_MODEL)
    reps = (s + CHUNK) // s + 1
    pe_ext = jnp.tile(pe, (reps, 1))[: s + CHUNK, :]
    out = _run(xf, pe_ext, table, n_rows, s, n_chunks)
    return out.reshape(b, s, D_MODEL)


# speedup vs baseline: 1.3478x; 1.0050x over previous
import functools

import jax
import jax.numpy as jnp
from jax import lax
from jax.experimental import pallas as pl
from jax.experimental.pallas import tpu as pltpu
from jax.experimental.pallas import tpu_sc as plsc

D_MODEL = 64
NC = 2
NS = 16
NW = NC * NS
BB = 128
NDT = D_MODEL // 8


def _positional_encoding(seq_len, d_model):
    pos = jnp.arange(0, seq_len, dtype=jnp.float32)[:, None]
    dim = jnp.arange(0, d_model, dtype=jnp.float32)
    result = jnp.zeros((seq_len, d_model), dtype=jnp.float32)
    sin_part = jnp.sin(pos / (10000.0 ** (dim[0::2] / d_model)))
    cos_part = jnp.cos(pos / (10000.0 ** (dim[1::2] / d_model)))
    result = result.at[:, 0::2].set(sin_part)
    result = result.at[:, 1::2].set(cos_part)
    return result


@functools.partial(jax.jit, static_argnames=("seq_len", "n_batch"))
def _run(xt, pe, table, seq_len, n_batch):
    mesh = plsc.VectorSubcoreMesh(
        core_axis_name="c", subcore_axis_name="s", num_cores=NC,
        num_subcores=NS)
    nbt = n_batch // BB
    n_pairs = seq_len // 2

    @functools.partial(
        pl.kernel,
        out_type=jax.ShapeDtypeStruct((seq_len, NDT, nbt, 8 * BB),
                                      jnp.float32),
        mesh=mesh,
        scratch_types=[
            pltpu.VMEM((seq_len, BB), jnp.int32),
            pltpu.VMEM((seq_len, D_MODEL), jnp.float32),
            pltpu.VMEM((BB, D_MODEL), jnp.float32),
            pltpu.VMEM((BB, D_MODEL), jnp.float32),
            pltpu.VMEM((NDT, 8 * BB), jnp.float32),
            pltpu.VMEM((NDT, 8 * BB), jnp.float32),
            pltpu.SemaphoreType.DMA,
            pltpu.SemaphoreType.DMA,
            pltpu.SemaphoreType.DMA,
            pltpu.SemaphoreType.DMA,
        ],
        compiler_params=pltpu.CompilerParams(
            use_tc_tiling_on_sc=False, needs_layout_passes=False),
    )
    def k(xt_hbm, pe_hbm, table_hbm, out_hbm, idx_v, pe_v, rb0, rb1, tb0,
          tb1, gsem0, gsem1, ssem0, ssem1):
        cid = lax.axis_index("c")
        sid = lax.axis_index("s")
        wid = sid * NC + cid
        rb = (rb0, rb1)
        tb = (tb0, tb1)
        gsem = (gsem0, gsem1)
        ssem = (ssem0, ssem1)

        pltpu.sync_copy(xt_hbm.at[:, pl.ds(wid * BB, BB)], idx_v)
        pltpu.sync_copy(pe_hbm, pe_v)

        lanes = lax.iota(jnp.int32, 16)
        dt_idx = [(lanes + 16 * j) >> 3 for j in range(4)]
        in_idx = [((lanes + 16 * j) & 7) * BB for j in range(4)]

        def fire(s, p):
            pltpu.async_copy(table_hbm.at[idx_v.at[s]], rb[p], gsem[p])

        def drain_gather(p):
            pltpu.make_async_copy(table_hbm.at[idx_v.at[0]], rb[p],
                                  gsem[p]).wait()

        def transform(s, p):
            pe_j = [pe_v[s, pl.ds(16 * j, 16)] for j in range(4)]
            rbp, tbp = rb[p], tb[p]

            @plsc.parallel_loop(0, BB, unroll=8)
            def _(b):
                for j in range(4):
                    v = rbp[b, pl.ds(16 * j, 16)] + pe_j[j]
                    plsc.store_scatter(tbp, [dt_idx[j], in_idx[j] + b], v)

        def store(s, p):
            pltpu.async_copy(tb[p], out_hbm.at[s, :, wid], ssem[p])

        def drain_store(p):
            pltpu.make_async_copy(tb[p], out_hbm.at[0, :, wid],
                                  ssem[p]).wait()

        fire(0, 0)

        def pair_body(kk, _):
            s0 = 2 * kk
            s1 = s0 + 1
            fire(s1, 1)
            drain_gather(0)

            @pl.when(kk > 0)
            def _():
                drain_store(0)

            transform(s0, 0)
            store(s0, 0)

            @pl.when(kk < n_pairs - 1)
            def _():
                fire(s0 + 2, 0)

            drain_gather(1)

            @pl.when(kk > 0)
            def _():
                drain_store(1)

            transform(s1, 1)
            store(s1, 1)
            return ()

        lax.fori_loop(0, n_pairs, pair_body, ())
        drain_store(0)
        drain_store(1)

    return k(xt, pe, table)


def kernel(x, table):
    b, s = x.shape
    xt = jnp.transpose(x).astype(jnp.int32)
    pe = _positional_encoding(s, D_MODEL)
    out4 = _run(xt, pe, table, s, b)
    out5 = out4.reshape(s, NDT, b // BB, 8, BB)
    return out5.transpose((2, 4, 0, 1, 3)).reshape(b, s, D_MODEL)


# speedup vs baseline: 2.1342x; 1.5834x over previous
import functools

import jax
import jax.numpy as jnp
from jax import lax
from jax.experimental import pallas as pl
from jax.experimental.pallas import tpu as pltpu
from jax.experimental.pallas import tpu_sc as plsc

D_MODEL = 64
NC = 2
NS = 16
NW = NC * NS
BB = 128
NDT = D_MODEL // 8


def _positional_encoding(seq_len, d_model):
    pos = jnp.arange(0, seq_len, dtype=jnp.float32)[:, None]
    dim = jnp.arange(0, d_model, dtype=jnp.float32)
    result = jnp.zeros((seq_len, d_model), dtype=jnp.float32)
    sin_part = jnp.sin(pos / (10000.0 ** (dim[0::2] / d_model)))
    cos_part = jnp.cos(pos / (10000.0 ** (dim[1::2] / d_model)))
    result = result.at[:, 0::2].set(sin_part)
    result = result.at[:, 1::2].set(cos_part)
    return result


@functools.partial(jax.jit, static_argnames=("seq_len", "n_batch"))
def _run(xt, pe, table, seq_len, n_batch):
    mesh = plsc.VectorSubcoreMesh(
        core_axis_name="c", subcore_axis_name="s", num_cores=NC,
        num_subcores=NS)
    nbt = n_batch // BB
    n_pairs = seq_len // 2

    @functools.partial(
        pl.kernel,
        out_type=jax.ShapeDtypeStruct((seq_len, NDT, nbt, 8, BB),
                                      jnp.float32),
        mesh=mesh,
        scratch_types=[
            pltpu.VMEM((seq_len, BB), jnp.int32),
            pltpu.VMEM((seq_len, D_MODEL), jnp.float32),
            pltpu.VMEM((BB, D_MODEL), jnp.float32),
            pltpu.VMEM((BB, D_MODEL), jnp.float32),
            pltpu.VMEM((NDT, 8, BB + 1), jnp.float32),
            pltpu.VMEM((NDT, 8, BB + 1), jnp.float32),
            pltpu.SemaphoreType.DMA,
            pltpu.SemaphoreType.DMA,
            pltpu.SemaphoreType.DMA,
            pltpu.SemaphoreType.DMA,
        ],
        compiler_params=pltpu.CompilerParams(
            use_tc_tiling_on_sc=False, needs_layout_passes=False),
    )
    def k(xt_hbm, pe_hbm, table_hbm, out_hbm, idx_v, pe_v, rb0, rb1, tb0,
          tb1, gsem0, gsem1, ssem0, ssem1):
        cid = lax.axis_index("c")
        sid = lax.axis_index("s")
        wid = sid * NC + cid
        rb = (rb0, rb1)
        tb = (tb0, tb1)
        gsem = (gsem0, gsem1)
        ssem = (ssem0, ssem1)

        pltpu.sync_copy(xt_hbm.at[:, pl.ds(wid * BB, BB)], idx_v)
        pltpu.sync_copy(pe_hbm, pe_v)

        lanes = lax.iota(jnp.int32, 16)
        dt_idx = [(lanes + 16 * j) >> 3 for j in range(4)]
        di_idx = [(lanes + 16 * j) & 7 for j in range(4)]

        def fire(s, p):
            pltpu.async_copy(table_hbm.at[idx_v.at[s]], rb[p], gsem[p])

        def drain_gather(p):
            pltpu.make_async_copy(table_hbm.at[idx_v.at[0]], rb[p],
                                  gsem[p]).wait()

        def transform(s, p):
            pe_j = [pe_v[s, pl.ds(16 * j, 16)] for j in range(4)]
            rbp, tbp = rb[p], tb[p]

            @plsc.parallel_loop(0, BB, unroll=8)
            def _(b):
                colv = (lanes & 0) + b
                for j in range(4):
                    v = rbp[b, pl.ds(16 * j, 16)] + pe_j[j]
                    plsc.store_scatter(tbp, [dt_idx[j], di_idx[j], colv], v)

        def store(s, p):
            pltpu.async_copy(tb[p].at[:, :, pl.ds(0, BB)],
                             out_hbm.at[s, :, wid], ssem[p])

        def drain_store(p):
            pltpu.make_async_copy(tb[p].at[:, :, pl.ds(0, BB)],
                                  out_hbm.at[0, :, wid], ssem[p]).wait()

        fire(0, 0)

        def pair_body(kk, _):
            s0 = 2 * kk
            s1 = s0 + 1
            fire(s1, 1)
            drain_gather(0)

            @pl.when(kk > 0)
            def _():
                drain_store(0)

            transform(s0, 0)
            store(s0, 0)

            @pl.when(kk < n_pairs - 1)
            def _():
                fire(s0 + 2, 0)

            drain_gather(1)

            @pl.when(kk > 0)
            def _():
                drain_store(1)

            transform(s1, 1)
            store(s1, 1)
            return ()

        lax.fori_loop(0, n_pairs, pair_body, ())
        drain_store(0)
        drain_store(1)

    return k(xt, pe, table)


def kernel(x, table):
    b, s = x.shape
    xt = jnp.transpose(x).astype(jnp.int32)
    pe = _positional_encoding(s, D_MODEL)
    out5 = _run(xt, pe, table, s, b)
    return out5.transpose((2, 4, 0, 1, 3)).reshape(b, s, D_MODEL)
